# baseline (speedup 1.0000x reference)
import jax
import jax.numpy as jnp
from jax import lax
from jax.experimental import pallas as pl

_N = 48
_E = _N * _N
_DH = 128
_DV = 64
_H = 8
_dh = _DH // _H
_dvh = _DV // _H
_NRBF = 16
_L = 4
_G = 8


def _se3_body(h_ref, xt_ref, win_ref, wqkv_ref, wo_ref,
              bo_ref, w1catt_ref, b1col_ref, wbdt_ref, repsel_ref,
              tilesel_ref, g_ref, b_ref, wfin_ref, bfin_ref, out_ref):
    f32 = jnp.float32
    ng = _G * _N
    sh = _H * _N
    centers_col = lax.broadcasted_iota(
        jnp.int32, (_NRBF, 1), 0).astype(f32) * (5.0 / (_NRBF - 1))
    si = lax.broadcasted_iota(jnp.int32, (_G * sh, _N), 0)
    sj = lax.broadcasted_iota(jnp.int32, (_G * sh, _N), 1)
    negdiag = jnp.where((si // _H) % _N == sj, -1e30, 0.0)
    hm0 = lax.broadcasted_iota(jnp.int32, (sh, _DH), 0)
    hm1 = lax.broadcasted_iota(jnp.int32, (sh, _DH), 1)
    headmask = (hm0 % _H == hm1 // _dh).astype(f32)
    sm0 = lax.broadcasted_iota(jnp.int32, (sh, _DV), 0)
    sm1 = lax.broadcasted_iota(jnp.int32, (sh, _DV), 1)
    selmask = (sm0 % _H == sm1 // _dvh).astype(f32)
    sg0 = lax.broadcasted_iota(jnp.int32, (_G, ng), 0)
    sg1 = lax.broadcasted_iota(jnp.int32, (_G, ng), 1)
    sumsel = (sg1 // _N == sg0).astype(f32)

    feats = jnp.dot(h_ref[...].reshape(ng, -1), win_ref[...],
                    preferred_element_type=f32)

    sts = []
    for g in range(_G):
        xtg = xt_ref[g]
        xit = jnp.dot(xtg, repsel_ref[...],
                      preferred_element_type=f32)
        xjt = jnp.dot(xtg, tilesel_ref[...],
                      preferred_element_type=f32)
        dt = xjt - xit
        d2t = jnp.sum(dt * dt, axis=0, keepdims=True)
        distt = jnp.sqrt(d2t + 1e-12)
        rbt = jnp.exp(-4.0 * (distt - centers_col) ** 2)
        rhidt = jnp.maximum(
            jnp.dot(w1catt_ref[...], rbt, preferred_element_type=f32)
            + b1col_ref[...], 0.0)
        rkvt = jnp.dot(wbdt_ref[...], rhidt,
                       preferred_element_type=f32)
        rkv3 = jnp.transpose(rkvt).reshape(_N, _N, _L * 2 * _H)
        sts.append(jnp.transpose(rkv3, (0, 2, 1)))

    for l in range(_L):
        qkv = jnp.dot(feats, wqkv_ref[l], preferred_element_type=f32)
        q = qkv[:, :_DH]
        k = qkv[:, _DH:2 * _DH]
        v = qkv[:, 2 * _DH:]

        lst = jnp.concatenate(
            [jnp.dot(
                jnp.broadcast_to(q[g * _N:(g + 1) * _N][:, None],
                                 (_N, _H, _DH)).reshape(sh, _DH) * headmask,
                jnp.transpose(k[g * _N:(g + 1) * _N]),
                preferred_element_type=f32) for g in range(_G)],
            axis=0)
        c0 = l * 2 * _H
        rkst = jnp.concatenate(
            [sts[g][:, c0:c0 + _H, :].reshape(sh, _N)
             for g in range(_G)], axis=0)
        rvst = jnp.concatenate(
            [sts[g][:, c0 + _H:c0 + 2 * _H, :].reshape(sh, _N)
             for g in range(_G)], axis=0)
        lst = lst * rkst + negdiag
        mx = jnp.max(lst, axis=1, keepdims=True)
        ex = jnp.exp(lst - mx)
        den = jnp.sum(ex, axis=1, keepdims=True)
        ast = ex / (den + 1e-9) * rvst

        agg = jnp.concatenate(
            [jnp.sum(
                (jnp.dot(ast[g * sh:(g + 1) * sh],
                         v[g * _N:(g + 1) * _N],
                         preferred_element_type=f32) * selmask
                 ).reshape(_N, _H, _DV),
                axis=1) for g in range(_G)],
            axis=0)

        feats = feats + jnp.dot(agg, wo_ref[l],
                                preferred_element_type=f32) + bo_ref[l]
        mu = jnp.mean(feats, axis=-1, keepdims=True)
        xc = feats - mu
        var = jnp.mean(xc * xc, axis=-1, keepdims=True)
        feats = xc / jnp.sqrt(var + 1e-5) * g_ref[l] + b_ref[l]

    out = jnp.dot(feats, wfin_ref[...], preferred_element_type=f32)
    out = out + bfin_ref[...]
    res = jnp.dot(sumsel, out, preferred_element_type=f32)
    for g in range(_G):
        out_ref[g] = res[g:g + 1]


def kernel(h, x, n_atoms, params):
    Bsz = h.shape[0]
    f32 = jnp.float32
    layers = params["layers"]

    def stk(name):
        return jnp.stack([p[name] for p in layers])

    wqkv = jnp.concatenate([stk("Wq"), stk("Wk"), stk("Wv")], axis=2)
    wo = stk("Wo")
    bo = stk("bo").reshape(_L, 1, _DH)
    w1cat = jnp.concatenate([p["Wr1"] for p in layers], axis=1)
    b1cat = jnp.concatenate([p["br1"] for p in layers]).reshape(1, -1)
    rhid_n = layers[0]["Wr1"].shape[1]
    wbd = jnp.zeros((_L * rhid_n, _L * 2 * _H), f32)
    inv_sqrt_dh = 1.0 / (_dh ** 0.5)
    for l, p in enumerate(layers):
        blk = jnp.concatenate([p["Wrk"] * inv_sqrt_dh, p["Wrv"]], axis=1)
        wbd = wbd.at[l * rhid_n:(l + 1) * rhid_n,
                     l * 2 * _H:(l + 1) * 2 * _H].set(blk)
    gamma = stk("gamma").reshape(_L, 1, _DH)
    beta = stk("beta").reshape(_L, 1, _DH)
    bfin = params["b_fin"].reshape(1, _DH)
    w1catt = w1cat.T
    b1col = b1cat.reshape(-1, 1)
    wbdt = wbd.T
    ev = jnp.arange(_E)
    av = jnp.arange(_N)[:, None]
    repsel = (ev[None, :] // _N == av).astype(f32)
    tilesel = (ev[None, :] % _N == av).astype(f32)

    def full(shape):
        return pl.BlockSpec(shape, lambda b: (0,) * len(shape))

    grid_spec = pl.GridSpec(
        grid=(Bsz // _G,),
        in_specs=[
            pl.BlockSpec((_G, _N, h.shape[2]), lambda b: (b, 0, 0)),
            pl.BlockSpec((_G, 3, _N), lambda b: (b, 0, 0)),
            full(params["W_in"].shape),
            full(wqkv.shape), full(wo.shape),
            full(bo.shape), full(w1catt.shape), full(b1col.shape),
            full(wbdt.shape), full(repsel.shape), full(tilesel.shape),
            full(gamma.shape),
            full(beta.shape), full(params["W_fin"].shape), full(bfin.shape),
        ],
        out_specs=pl.BlockSpec((_G, 1, _DH), lambda b: (b, 0, 0)),
    )

    out = pl.pallas_call(
        _se3_body,
        grid_spec=grid_spec,
        out_shape=jax.ShapeDtypeStruct((Bsz, 1, _DH), f32),
    )(h, jnp.swapaxes(x, 1, 2), params["W_in"], wqkv, wo, bo,
      w1catt, b1col, wbdt, repsel, tilesel,
      gamma, beta, params["W_fin"], bfin)

    return out.reshape(Bsz, _DH) / jnp.asarray(n_atoms, f32)
